# baseline (speedup 1.0000x reference)
import functools

import jax
import jax.numpy as jnp
from jax import lax
from jax.experimental import pallas as pl
from jax.experimental.pallas import tpu as pltpu
from jax.experimental.pallas import tpu_sc as plsc


def _sc_geometry():
    try:
        info = plsc.get_sparse_core_info()
        return info.num_cores, info.num_subcores
    except Exception:
        return 2, 16


def kernel(embedding_ids, embedding_table):
    B, H = embedding_ids.shape
    V, D = embedding_table.shape
    NC, NS = _sc_geometry()
    NW = NC * NS

    rows_w = B // NW
    HPC = 2
    CH = HPC * rows_w
    n_chunks = H // HPC
    RING = 3
    LOOKAHEAD = 2
    assert rows_w * NW == B and n_chunks * HPC == H

    idx = embedding_ids.astype(jnp.int32).T.reshape(H, NW, rows_w)

    mesh = plsc.VectorSubcoreMesh(core_axis_name="c", subcore_axis_name="s")

    @functools.partial(
        pl.kernel,
        out_type=jax.ShapeDtypeStruct((B, H * D), jnp.float32),
        mesh=mesh,
        scratch_types=[
            pltpu.VMEM((H, rows_w), jnp.int32),
            pltpu.VMEM((RING, HPC, rows_w, D), jnp.float32),
        ]
        + [pltpu.SemaphoreType.DMA] * (2 * RING),
    )
    def run(idx_hbm, tbl_hbm, out_hbm, idx_v, rows, *sems):
        gsem = sems[:RING]
        wsem = sems[RING:]
        wid = lax.axis_index("s") * NC + lax.axis_index("c")
        row0 = wid * rows_w
        pltpu.sync_copy(
            idx_hbm.at[pl.ds(0, H), pl.ds(wid, 1)],
            idx_v.reshape(H, 1, rows_w),
        )
        idx2 = idx_v.reshape(n_chunks, CH)

        def gather(j, b):
            pltpu.async_copy(
                tbl_hbm.at[idx2.at[j]], rows.at[b].reshape(CH, D), gsem[b]
            )

        def wait_gather(b):
            pltpu.make_async_copy(
                tbl_hbm.at[idx2.at[0]], rows.at[b].reshape(CH, D), gsem[b]
            ).wait()

        def write(j, b):
            for c in range(HPC):
                pltpu.async_copy(
                    rows.at[b].at[c],
                    out_hbm.at[
                        pl.ds(row0, rows_w), pl.ds((j * HPC + c) * D, D)
                    ],
                    wsem[b],
                )

        def wait_write(b):
            for c in range(HPC):
                pltpu.make_async_copy(
                    rows.at[b].at[c],
                    out_hbm.at[pl.ds(row0, rows_w), pl.ds(0, D)],
                    wsem[b],
                ).wait()

        for b in range(LOOKAHEAD):
            gather(b, b)

        n_steady = n_chunks - 1
        assert n_steady % RING == 0

        def body(p, carry):
            for b in range(RING):
                j = RING * p + b
                bp = (b + LOOKAHEAD) % RING
                if b == 0:

                    @pl.when(p > 0)
                    def _(bp=bp):
                        wait_write(bp)

                else:
                    wait_write(bp)
                max_j = RING * (n_steady // RING - 1) + b
                if max_j + LOOKAHEAD < n_chunks:
                    gather(j + LOOKAHEAD, bp)
                else:

                    @pl.when(j + LOOKAHEAD < n_chunks)
                    def _(j=j, bp=bp):
                        gather(j + LOOKAHEAD, bp)

                wait_gather(b)
                write(j, b)
            return carry

        lax.fori_loop(0, n_steady // RING, body, 0)

        last = n_chunks - 1
        lb = last % RING
        wait_write((lb + LOOKAHEAD) % RING)
        wait_gather(lb)
        write(last, lb)
        wait_write(lb)

    return run(idx, embedding_table)


# speedup vs baseline: 1.0057x; 1.0057x over previous
import functools

import jax
import jax.numpy as jnp
from jax import lax
from jax.experimental import pallas as pl
from jax.experimental.pallas import tpu as pltpu
from jax.experimental.pallas import tpu_sc as plsc


def _sc_geometry():
    try:
        info = plsc.get_sparse_core_info()
        return info.num_cores, info.num_subcores
    except Exception:
        return 2, 16


def kernel(embedding_ids, embedding_table):
    B, H = embedding_ids.shape
    V, D = embedding_table.shape
    NC, NS = _sc_geometry()
    NW = NC * NS

    rows_w = B // NW
    n_chunks = H
    RING = 5
    LOOKAHEAD = 2
    assert rows_w * NW == B
    assert n_chunks % RING == 0 and n_chunks >= RING

    idx = embedding_ids.astype(jnp.int32).T.reshape(H, NW, rows_w)

    mesh = plsc.VectorSubcoreMesh(core_axis_name="c", subcore_axis_name="s")

    @functools.partial(
        pl.kernel,
        out_type=jax.ShapeDtypeStruct((B, H * D), jnp.float32),
        mesh=mesh,
        scratch_types=[
            pltpu.VMEM((n_chunks, rows_w), jnp.int32),
            pltpu.VMEM((RING, rows_w, D), jnp.float32),
        ]
        + [pltpu.SemaphoreType.DMA] * (2 * RING),
    )
    def run(idx_hbm, tbl_hbm, out_hbm, idx_v, rows, *sems):
        gsem = sems[:RING]
        wsem = sems[RING:]
        wid = lax.axis_index("s") * NC + lax.axis_index("c")
        row0 = wid * rows_w
        pltpu.sync_copy(
            idx_hbm.at[pl.ds(0, n_chunks), pl.ds(wid, 1)],
            idx_v.reshape(n_chunks, 1, rows_w),
        )

        def gather(j, b):
            pltpu.async_copy(
                tbl_hbm.at[idx_v.at[j]], rows.at[b], gsem[b]
            )

        def wait_gather(b):
            pltpu.make_async_copy(
                tbl_hbm.at[idx_v.at[0]], rows.at[b], gsem[b]
            ).wait()

        def write(j, b):
            pltpu.async_copy(
                rows.at[b],
                out_hbm.at[pl.ds(row0, rows_w), pl.ds(j * D, D)],
                wsem[b],
            )

        def wait_write(b):
            pltpu.make_async_copy(
                rows.at[b],
                out_hbm.at[pl.ds(row0, rows_w), pl.ds(0, D)],
                wsem[b],
            ).wait()

        for b in range(LOOKAHEAD):
            gather(b, b)

        def body(p, carry):
            for b in range(RING):
                j = RING * p + b
                bp = (b + LOOKAHEAD) % RING
                if b < RING - LOOKAHEAD:

                    @pl.when(p > 0)
                    def _(bp=bp):
                        wait_write(bp)

                else:
                    wait_write(bp)
                max_j = RING * (n_chunks // RING - 1) + b
                if max_j + LOOKAHEAD < n_chunks:
                    gather(j + LOOKAHEAD, bp)
                else:

                    @pl.when(j + LOOKAHEAD < n_chunks)
                    def _(j=j, bp=bp):
                        gather(j + LOOKAHEAD, bp)

                wait_gather(b)
                write(j, b)
            return carry

        lax.fori_loop(0, n_chunks // RING, body, 0)

        for j in range(n_chunks - (RING - LOOKAHEAD), n_chunks):
            wait_write(j % RING)

    return run(idx, embedding_table)
